# baseline (speedup 1.0000x reference)
import math

import jax
import jax.numpy as jnp
from jax import lax
from jax.experimental import pallas as pl
from jax.experimental.pallas import tpu as pltpu
from jax.experimental.pallas import tpu_sc as plsc

D = 16
R = 1
N_NODES = 100000
N_MOTIFS = 1600000
NUM_TAU = 16
NUM_GRAPHS = 8

NC = 2
NS = 16
NW = NC * NS
B = 512
NSUB = B // 128
NSLOT = 4
CHUNKS = 100
M_PAD = NW * B * CHUNKS
GROUPS = B // 16

S_ACC = 100352
SLICE = S_ACC // NS
DUMP = N_NODES

LAMBDA_3 = math.log1p(math.exp(0.5))
BETA_3 = min(math.log1p(math.exp(1.0)), 5.0)
COEF = BETA_3 / math.sqrt(R * D)
OUT_SCALE = LAMBDA_3 / BETA_3

_GATHER_DNUMS = lax.GatherDimensionNumbers(
    offset_dims=(), collapsed_slice_dims=(0,), start_index_map=(0,))


def _shuf(x, idx):
    return lax.gather(x, idx[:, None], _GATHER_DNUMS, (1,),
                      mode=lax.GatherScatterMode.PROMISE_IN_BOUNDS)


def _sc_body(x_hbm, q_hbm, k_hbm, tt_hbm, z_hbm, out_hbm,
             i_v, q_r, ku_r, kv_r, e_v, csc, tt_v, acc_sp, gsem, ssem):
    cid = lax.axis_index("c")
    sid = lax.axis_index("s")
    wid = sid * NC + cid
    base = wid * CHUNKS

    pltpu.sync_copy(z_hbm, acc_sp.at[pl.ds(sid * SLICE, SLICE)])
    pltpu.sync_copy(tt_hbm, tt_v)
    plsc.subcore_barrier()

    lane = lax.iota(jnp.int32, 16)
    _XOR_PERMS = {k: jnp.bitwise_xor(lane, k) for k in (1, 2, 4, 8)}

    def fetch(k, s):
        pltpu.sync_copy(x_hbm.at[k], i_v[s])
        for j in range(NSUB):
            dst = pl.ds(j * 128, 128)
            idx = pl.ds(j * 128, 128)
            pltpu.async_copy(q_hbm.at[i_v[s].at[0, idx]], q_r[s].at[dst],
                             gsem[s])
            pltpu.async_copy(k_hbm.at[i_v[s].at[1, idx]], ku_r[s].at[dst],
                             gsem[s])
            pltpu.async_copy(k_hbm.at[i_v[s].at[2, idx]], kv_r[s].at[dst],
                             gsem[s])

    def wait_gathers(s):
        for j in range(NSUB):
            dst = pl.ds(j * 128, 128)
            idx = pl.ds(j * 128, 128)
            pltpu.make_async_copy(q_hbm.at[i_v[s].at[0, idx]],
                                  q_r[s].at[dst], gsem[s]).wait()
            pltpu.make_async_copy(k_hbm.at[i_v[s].at[1, idx]],
                                  ku_r[s].at[dst], gsem[s]).wait()
            pltpu.make_async_copy(k_hbm.at[i_v[s].at[2, idx]],
                                  kv_r[s].at[dst], gsem[s]).wait()

    def start_scatter(s):
        for j in range(NSUB):
            pltpu.async_copy(e_v[s].at[j], acc_sp.at[csc[s].at[j]],
                             ssem[s], add=True)

    def drain_scatter(s):
        for j in range(NSUB):
            pltpu.make_async_copy(e_v[s].at[j], acc_sp.at[csc[s].at[j]],
                                  ssem[s]).wait()

    def compute(s):
        qr, kur, kvr, iv, ev = q_r[s], ku_r[s], kv_r[s], i_v[s], e_v[s]

        def group_body(g, carry2):
            m0 = g * 16
            t16 = iv[3, pl.ds(m0, 16)]
            bvec = jnp.zeros((16,), jnp.float32)
            for i in range(16):
                m = m0 + i
                trow = tt_v[t16[i], :]
                w = qr[m, :] * (kur[m, :] * kvr[m, :] + trow)
                for k in (1, 2, 4, 8):
                    w = w + _shuf(w, _XOR_PERMS[k])
                bvec = jnp.where(lane == i, w, bvec)
            e = jnp.exp(bvec * COEF)
            erow = lax.shift_right_logical(g, 3)
            ecol = lax.mul(lax.bitwise_and(g, 7), 16)
            ev[erow, pl.ds(ecol, 16)] = e
            return carry2

        lax.fori_loop(0, GROUPS, group_body, 0)

    for s in range(NSLOT - 1):
        fetch(base + s, s)

    def outer_body(it, carry):
        for s in range(NSLOT):
            cc = it * NSLOT + s
            wait_gathers(s)

            @pl.when(it > 0)
            def _():
                drain_scatter(s)

            for j in range(NSUB):
                for l in range(8):
                    csc[s][j, pl.ds(l * 16, 16)] = (
                        i_v[s][0, pl.ds(j * 128 + l * 16, 16)])
            compute(s)
            start_scatter(s)

            @pl.when(cc + NSLOT - 1 < CHUNKS)
            def _():
                fetch(base + cc + NSLOT - 1, (s + NSLOT - 1) % NSLOT)
        return carry

    lax.fori_loop(0, CHUNKS // NSLOT, outer_body, 0)
    for s in range(NSLOT):
        drain_scatter(s)

    plsc.subcore_barrier()
    pltpu.sync_copy(acc_sp.at[pl.ds(sid * SLICE, SLICE)],
                    out_hbm.at[cid, pl.ds(sid * SLICE, SLICE)])


def _tc_finish_body(s0_ref, s1_ref, b_ref, o_ref):
    s = s0_ref[...] + s1_ref[...]
    lse = jnp.where(s > 0.0, jnp.log(s), 0.0)
    for g in range(NUM_GRAPHS):
        eg = jnp.sum(jnp.where(b_ref[...] == g, lse, 0.0))
        o_ref[g] = eg * OUT_SCALE


def kernel(G, c_3, u_3, v_3, t_tau, batch, num_graphs, Q3, K3, T_params,
           num_nodes):
    del G, num_graphs, num_nodes
    pad = M_PAD - N_MOTIFS
    i32 = jnp.int32
    nchunks = NW * CHUNKS
    c_p = jnp.concatenate([c_3.astype(i32),
                           jnp.full((pad,), DUMP, i32)]).reshape(nchunks, 1, B)
    u_p = jnp.concatenate([u_3.astype(i32),
                           jnp.zeros((pad,), i32)]).reshape(nchunks, 1, B)
    v_p = jnp.concatenate([v_3.astype(i32),
                           jnp.zeros((pad,), i32)]).reshape(nchunks, 1, B)
    t_p = jnp.concatenate([t_tau.astype(i32),
                           jnp.zeros((pad,), i32)]).reshape(nchunks, 1, B)
    x = jnp.concatenate([c_p, u_p, v_p, t_p], axis=1)
    q2 = Q3.reshape(N_NODES, R * D)
    k2 = K3.reshape(N_NODES, R * D)
    tt = T_params.reshape(NUM_TAU, R * D)
    zeros = jnp.zeros((SLICE,), jnp.float32)

    mesh = plsc.VectorSubcoreMesh(core_axis_name="c", subcore_axis_name="s")
    sc = pl.kernel(
        _sc_body,
        out_type=jax.ShapeDtypeStruct((NC, S_ACC), jnp.float32),
        mesh=mesh,
        scratch_types=[
            [pltpu.VMEM((4, B), i32) for _ in range(NSLOT)],
            [pltpu.VMEM((B, R * D), jnp.float32) for _ in range(NSLOT)],
            [pltpu.VMEM((B, R * D), jnp.float32) for _ in range(NSLOT)],
            [pltpu.VMEM((B, R * D), jnp.float32) for _ in range(NSLOT)],
            [pltpu.VMEM((NSUB, 128), jnp.float32) for _ in range(NSLOT)],
            [pltpu.VMEM((NSUB, 128), i32) for _ in range(NSLOT)],
            pltpu.VMEM((NUM_TAU, R * D), jnp.float32),
            pltpu.VMEM_SHARED((S_ACC,), jnp.float32),
            [pltpu.SemaphoreType.DMA for _ in range(NSLOT)],
            [pltpu.SemaphoreType.DMA for _ in range(NSLOT)],
        ],
        compiler_params=pltpu.CompilerParams(
            needs_layout_passes=False, use_tc_tiling_on_sc=False),
    )
    partials = sc(x, q2, k2, tt, zeros)

    batch_pad = jnp.concatenate(
        [batch.astype(i32), jnp.full((S_ACC - N_NODES,), NUM_GRAPHS, i32)]
    ).reshape(-1, 128)
    s0 = partials[0].reshape(-1, 128)
    s1 = partials[1].reshape(-1, 128)

    out = pl.pallas_call(
        _tc_finish_body,
        out_shape=jax.ShapeDtypeStruct((NUM_GRAPHS,), jnp.float32),
        out_specs=pl.BlockSpec(memory_space=pltpu.SMEM),
    )(s0, s1, batch_pad)
    return out


# speedup vs baseline: 1.5879x; 1.5879x over previous
import math

import jax
import jax.numpy as jnp
from jax import lax
from jax.experimental import pallas as pl
from jax.experimental.pallas import tpu as pltpu
from jax.experimental.pallas import tpu_sc as plsc

D = 16
R = 1
N_NODES = 100000
N_MOTIFS = 1600000
NUM_TAU = 16
NUM_GRAPHS = 8

NC = 2
NS = 16
NW = NC * NS
MW = N_MOTIFS // NW
B = 512
NSUB = B // 128
NSLOT = 4
C_RING = 96
TAIL = MW - (C_RING + 1) * B
TGROUPS = TAIL // 16
GROUPS = B // 16

S_ACC = 100352
SLICE = S_ACC // NS
DUMP = N_NODES

LAMBDA_3 = math.log1p(math.exp(0.5))
BETA_3 = min(math.log1p(math.exp(1.0)), 5.0)
COEF = BETA_3 / math.sqrt(R * D)
OUT_SCALE = LAMBDA_3 / BETA_3


def _sc_body(c_hbm, u_hbm, v_hbm, t_hbm, q_hbm, k_hbm, tt_hbm, z_hbm,
             out_hbm, i_v, q_r, ku_r, kv_r, e_v, csc, tt_v, acc_sp,
             gsem, ssem, isem):
    cid = lax.axis_index("c")
    sid = lax.axis_index("s")
    wid = sid * NC + cid
    mbase = wid * MW

    pltpu.sync_copy(z_hbm, acc_sp.at[pl.ds(sid * SLICE, SLICE)])
    pltpu.sync_copy(tt_hbm, tt_v)
    plsc.subcore_barrier()

    lane = lax.iota(jnp.int32, 16)

    def idx_copies(k, s, n):
        src = pl.ds(mbase + k * B, n)
        dst = pl.ds(0, n)
        return [
            pltpu.make_async_copy(c_hbm.at[src], i_v[s].at[0, dst], isem[s]),
            pltpu.make_async_copy(u_hbm.at[src], i_v[s].at[1, dst], isem[s]),
            pltpu.make_async_copy(v_hbm.at[src], i_v[s].at[2, dst], isem[s]),
            pltpu.make_async_copy(t_hbm.at[src], i_v[s].at[3, dst], isem[s]),
        ]

    def start_idx(k, s, n=B):
        for cp in idx_copies(k, s, n):
            cp.start()

    def wait_idx(k, s, n=B):
        for cp in idx_copies(k, s, n):
            cp.wait()

    def gather_copies(s):
        cps = []
        for j in range(NSUB):
            blk = pl.ds(j * 128, 128)
            cps.append(pltpu.make_async_copy(
                q_hbm.at[i_v[s].at[0, blk]], q_r[s].at[blk], gsem[s]))
            cps.append(pltpu.make_async_copy(
                k_hbm.at[i_v[s].at[1, blk]], ku_r[s].at[blk], gsem[s]))
            cps.append(pltpu.make_async_copy(
                k_hbm.at[i_v[s].at[2, blk]], kv_r[s].at[blk], gsem[s]))
        return cps

    def fire_gathers(s):
        for cp in gather_copies(s):
            cp.start()

    def wait_gathers(s):
        for cp in gather_copies(s):
            cp.wait()

    def scatter_copies(s):
        return [pltpu.make_async_copy(e_v[s].at[j], acc_sp.at[csc[s].at[j]],
                                      ssem[s])
                for j in range(NSUB)]

    def start_scatter(s):
        for cp in scatter_copies(s):
            cp.start(add=True)

    def drain_scatter(s):
        for cp in scatter_copies(s):
            cp.wait()

    def snapshot_c(s, ngroups):
        for p in range(ngroups):
            csc[s][p >> 3, pl.ds((p & 7) * 16, 16)] = (
                i_v[s][0, pl.ds(p * 16, 16)])
        dump = jnp.full((16,), DUMP, jnp.int32)
        for p in range(ngroups, GROUPS):
            csc[s][p >> 3, pl.ds((p & 7) * 16, 16)] = dump

    def compute(s, ngroups):
        qr, kur, kvr, iv, ev = q_r[s], ku_r[s], kv_r[s], i_v[s], e_v[s]

        def group_body(g, carry2):
            m0 = g * 16
            t16 = iv[3, pl.ds(m0, 16)]
            bvec = jnp.zeros((16,), jnp.float32)
            for i in range(16):
                m = m0 + i
                trow = tt_v[t16[i], :]
                w = qr[m, :] * (kur[m, :] * kvr[m, :] + trow)
                bvec = jnp.where(lane == i, jnp.sum(w), bvec)
            e = jnp.exp(bvec * COEF)
            erow = lax.shift_right_logical(g, 3)
            ecol = lax.mul(lax.bitwise_and(g, 7), 16)
            ev[erow, pl.ds(ecol, 16)] = e
            return carry2

        lax.fori_loop(0, ngroups, group_body, 0)

    start_idx(0, 0)
    wait_idx(0, 0)
    fire_gathers(0)
    start_idx(1, 1)

    def outer_body(it, carry):
        for s in range(NSLOT):
            cc = it * NSLOT + s
            wait_gathers(s)

            @pl.when(it > 0)
            def _():
                drain_scatter(s)

            snapshot_c(s, GROUPS)
            compute(s, GROUPS)
            start_scatter(s)

            @pl.when(cc + 2 < C_RING)
            def _():
                start_idx(cc + 2, (s + 2) % NSLOT)

            @pl.when(cc + 1 < C_RING)
            def _():
                wait_idx(cc + 1, (s + 1) % NSLOT)
                fire_gathers((s + 1) % NSLOT)
        return carry

    lax.fori_loop(0, C_RING // NSLOT, outer_body, 0)
    for s in range(NSLOT):
        drain_scatter(s)

    start_idx(C_RING, 0)
    wait_idx(C_RING, 0)
    fire_gathers(0)
    wait_gathers(0)
    snapshot_c(0, GROUPS)
    compute(0, GROUPS)
    start_scatter(0)

    start_idx(C_RING + 1, 1, TAIL)
    wait_idx(C_RING + 1, 1, TAIL)
    fire_gathers(1)
    wait_gathers(1)
    snapshot_c(1, TGROUPS)
    compute(1, TGROUPS)
    start_scatter(1)

    drain_scatter(0)
    drain_scatter(1)

    plsc.subcore_barrier()
    pltpu.sync_copy(acc_sp.at[pl.ds(sid * SLICE, SLICE)],
                    out_hbm.at[pl.ds(cid * S_ACC + sid * SLICE, SLICE)])


def _tc_finish_body(s0_ref, s1_ref, b_ref, o_ref):
    s = s0_ref[...] + s1_ref[...]
    lse = jnp.where(s > 0.0, jnp.log(s), 0.0)
    for g in range(NUM_GRAPHS):
        eg = jnp.sum(jnp.where(b_ref[...] == g, lse, 0.0))
        o_ref[g] = eg * OUT_SCALE


def kernel(G, c_3, u_3, v_3, t_tau, batch, num_graphs, Q3, K3, T_params,
           num_nodes):
    del G, num_graphs, num_nodes
    i32 = jnp.int32
    c_p = c_3.astype(i32)
    u_p = u_3.astype(i32)
    v_p = v_3.astype(i32)
    t_p = t_tau.astype(i32)
    q2 = Q3.reshape(N_NODES, R * D)
    k2 = K3.reshape(N_NODES, R * D)
    tt = T_params.reshape(NUM_TAU, R * D)
    zeros = jnp.zeros((SLICE,), jnp.float32)

    mesh = plsc.VectorSubcoreMesh(core_axis_name="c", subcore_axis_name="s")
    sc = pl.kernel(
        _sc_body,
        out_type=jax.ShapeDtypeStruct((NC * S_ACC,), jnp.float32),
        mesh=mesh,
        scratch_types=[
            [pltpu.VMEM((4, B), i32) for _ in range(NSLOT)],
            [pltpu.VMEM((B, R * D), jnp.float32) for _ in range(NSLOT)],
            [pltpu.VMEM((B, R * D), jnp.float32) for _ in range(NSLOT)],
            [pltpu.VMEM((B, R * D), jnp.float32) for _ in range(NSLOT)],
            [pltpu.VMEM((NSUB, 128), jnp.float32) for _ in range(NSLOT)],
            [pltpu.VMEM((NSUB, 128), i32) for _ in range(NSLOT)],
            pltpu.VMEM((NUM_TAU, R * D), jnp.float32),
            pltpu.VMEM_SHARED((S_ACC,), jnp.float32),
            [pltpu.SemaphoreType.DMA for _ in range(NSLOT)],
            [pltpu.SemaphoreType.DMA for _ in range(NSLOT)],
            [pltpu.SemaphoreType.DMA for _ in range(NSLOT)],
        ],
        compiler_params=pltpu.CompilerParams(
            needs_layout_passes=False, use_tc_tiling_on_sc=False),
    )
    partials = sc(c_p, u_p, v_p, t_p, q2, k2, tt, zeros).reshape(NC, S_ACC)

    batch_pad = jnp.concatenate(
        [batch.astype(i32), jnp.full((S_ACC - N_NODES,), NUM_GRAPHS, i32)]
    ).reshape(-1, 128)
    s0 = partials[0].reshape(-1, 128)
    s1 = partials[1].reshape(-1, 128)

    out = pl.pallas_call(
        _tc_finish_body,
        out_shape=jax.ShapeDtypeStruct((NUM_GRAPHS,), jnp.float32),
        out_specs=pl.BlockSpec(memory_space=pltpu.SMEM),
    )(s0, s1, batch_pad)
    return out


# speedup vs baseline: 1.6935x; 1.0664x over previous
import math

import jax
import jax.numpy as jnp
from jax import lax
from jax.experimental import pallas as pl
from jax.experimental.pallas import tpu as pltpu
from jax.experimental.pallas import tpu_sc as plsc

D = 16
R = 1
N_NODES = 100000
N_MOTIFS = 1600000
NUM_TAU = 16
NUM_GRAPHS = 8

NC = 2
NS = 16
NW = NC * NS
MW = N_MOTIFS // NW
B = 1024
NSUB = B // 128
NSLOT = 2
C_RING = 48
TAIL = MW - C_RING * B
TGROUPS = TAIL // 16
GROUPS = B // 16

S_ACC = 100352
SLICE = S_ACC // NS
DUMP = N_NODES

LAMBDA_3 = math.log1p(math.exp(0.5))
BETA_3 = min(math.log1p(math.exp(1.0)), 5.0)
COEF = BETA_3 / math.sqrt(R * D)
OUT_SCALE = LAMBDA_3 / BETA_3


def _sc_body(c_hbm, u_hbm, v_hbm, t_hbm, q_hbm, k_hbm, tt_hbm, z_hbm,
             out_hbm, i_v, q_r, ku_r, kv_r, e_v, csc, tt_v, acc_sp,
             gsem, ssem, isem):
    cid = lax.axis_index("c")
    sid = lax.axis_index("s")
    wid = sid * NC + cid
    mbase = wid * MW

    pltpu.sync_copy(z_hbm, acc_sp.at[pl.ds(sid * SLICE, SLICE)])
    pltpu.sync_copy(tt_hbm, tt_v)
    plsc.subcore_barrier()

    lane = lax.iota(jnp.int32, 16)

    def idx_copies(k, s, n):
        src = pl.ds(mbase + k * B, n)
        dst = pl.ds(0, n)
        return [
            pltpu.make_async_copy(c_hbm.at[src], i_v[s].at[0, dst], isem[s]),
            pltpu.make_async_copy(u_hbm.at[src], i_v[s].at[1, dst], isem[s]),
            pltpu.make_async_copy(v_hbm.at[src], i_v[s].at[2, dst], isem[s]),
            pltpu.make_async_copy(t_hbm.at[src], i_v[s].at[3, dst], isem[s]),
        ]

    def start_idx(k, s, n=B):
        for cp in idx_copies(k, s, n):
            cp.start()

    def wait_idx(k, s, n=B):
        for cp in idx_copies(k, s, n):
            cp.wait()

    def gather_copies(s):
        cps = []
        for j in range(NSUB):
            blk = pl.ds(j * 128, 128)
            cps.append(pltpu.make_async_copy(
                q_hbm.at[i_v[s].at[0, blk]], q_r[s].at[blk], gsem[s]))
            cps.append(pltpu.make_async_copy(
                k_hbm.at[i_v[s].at[1, blk]], ku_r[s].at[blk], gsem[s]))
            cps.append(pltpu.make_async_copy(
                k_hbm.at[i_v[s].at[2, blk]], kv_r[s].at[blk], gsem[s]))
        return cps

    def fire_gathers(s):
        for cp in gather_copies(s):
            cp.start()

    def wait_gathers(s):
        for cp in gather_copies(s):
            cp.wait()

    def scatter_copies(s):
        return [pltpu.make_async_copy(e_v[s].at[j], acc_sp.at[csc[s].at[j]],
                                      ssem[s])
                for j in range(NSUB)]

    def start_scatter(s):
        for cp in scatter_copies(s):
            cp.start(add=True)

    def drain_scatter(s):
        for cp in scatter_copies(s):
            cp.wait()

    def snapshot_c(s, ngroups):
        for p in range(ngroups):
            csc[s][p >> 3, pl.ds((p & 7) * 16, 16)] = (
                i_v[s][0, pl.ds(p * 16, 16)])
        dump = jnp.full((16,), DUMP, jnp.int32)
        for p in range(ngroups, GROUPS):
            csc[s][p >> 3, pl.ds((p & 7) * 16, 16)] = dump

    def compute(s, ngroups):
        qr, kur, kvr, iv, ev = q_r[s], ku_r[s], kv_r[s], i_v[s], e_v[s]

        def group_body(g, carry2):
            m0 = g * 16
            t16 = iv[3, pl.ds(m0, 16)]
            bvec = jnp.zeros((16,), jnp.float32)
            for i in range(16):
                m = m0 + i
                trow = tt_v[t16[i], :]
                w = qr[m, :] * (kur[m, :] * kvr[m, :] + trow)
                bvec = jnp.where(lane == i, jnp.sum(w), bvec)
            e = jnp.exp(bvec * COEF)
            erow = lax.shift_right_logical(g, 3)
            ecol = lax.mul(lax.bitwise_and(g, 7), 16)
            ev[erow, pl.ds(ecol, 16)] = e
            return carry2

        lax.fori_loop(0, ngroups, group_body, 0)

    start_idx(0, 0)
    wait_idx(0, 0)
    fire_gathers(0)
    start_idx(1, 1)

    def outer_body(it, carry):
        for s in range(NSLOT):
            cc = it * NSLOT + s
            wait_gathers(s)

            @pl.when(it > 0)
            def _():
                drain_scatter(s)

            snapshot_c(s, GROUPS)
            compute(s, GROUPS)
            start_scatter(s)

            @pl.when(cc + 2 < C_RING)
            def _():
                start_idx(cc + 2, (s + 2) % NSLOT)

            @pl.when(cc + 1 < C_RING)
            def _():
                wait_idx(cc + 1, (s + 1) % NSLOT)
                fire_gathers((s + 1) % NSLOT)
        return carry

    lax.fori_loop(0, C_RING // NSLOT, outer_body, 0)
    for s in range(NSLOT):
        drain_scatter(s)

    start_idx(C_RING, 0, TAIL)
    wait_idx(C_RING, 0, TAIL)
    fire_gathers(0)
    wait_gathers(0)
    snapshot_c(0, TGROUPS)
    compute(0, TGROUPS)
    start_scatter(0)
    drain_scatter(0)

    plsc.subcore_barrier()
    pltpu.sync_copy(acc_sp.at[pl.ds(sid * SLICE, SLICE)],
                    out_hbm.at[pl.ds(cid * S_ACC + sid * SLICE, SLICE)])


def _tc_finish_body(s0_ref, s1_ref, b_ref, o_ref):
    s = s0_ref[...] + s1_ref[...]
    lse = jnp.where(s > 0.0, jnp.log(s), 0.0)
    for g in range(NUM_GRAPHS):
        eg = jnp.sum(jnp.where(b_ref[...] == g, lse, 0.0))
        o_ref[g] = eg * OUT_SCALE


def kernel(G, c_3, u_3, v_3, t_tau, batch, num_graphs, Q3, K3, T_params,
           num_nodes):
    del G, num_graphs, num_nodes
    i32 = jnp.int32
    c_p = c_3.astype(i32)
    u_p = u_3.astype(i32)
    v_p = v_3.astype(i32)
    t_p = t_tau.astype(i32)
    q2 = Q3.reshape(N_NODES, R * D)
    k2 = K3.reshape(N_NODES, R * D)
    tt = T_params.reshape(NUM_TAU, R * D)
    zeros = jnp.zeros((SLICE,), jnp.float32)

    mesh = plsc.VectorSubcoreMesh(core_axis_name="c", subcore_axis_name="s")
    sc = pl.kernel(
        _sc_body,
        out_type=jax.ShapeDtypeStruct((NC * S_ACC,), jnp.float32),
        mesh=mesh,
        scratch_types=[
            [pltpu.VMEM((4, B), i32) for _ in range(NSLOT)],
            [pltpu.VMEM((B, R * D), jnp.float32) for _ in range(NSLOT)],
            [pltpu.VMEM((B, R * D), jnp.float32) for _ in range(NSLOT)],
            [pltpu.VMEM((B, R * D), jnp.float32) for _ in range(NSLOT)],
            [pltpu.VMEM((NSUB, 128), jnp.float32) for _ in range(NSLOT)],
            [pltpu.VMEM((NSUB, 128), i32) for _ in range(NSLOT)],
            pltpu.VMEM((NUM_TAU, R * D), jnp.float32),
            pltpu.VMEM_SHARED((S_ACC,), jnp.float32),
            [pltpu.SemaphoreType.DMA for _ in range(NSLOT)],
            [pltpu.SemaphoreType.DMA for _ in range(NSLOT)],
            [pltpu.SemaphoreType.DMA for _ in range(NSLOT)],
        ],
        compiler_params=pltpu.CompilerParams(
            needs_layout_passes=False, use_tc_tiling_on_sc=False),
    )
    partials = sc(c_p, u_p, v_p, t_p, q2, k2, tt, zeros).reshape(NC, S_ACC)

    batch_pad = jnp.concatenate(
        [batch.astype(i32), jnp.full((S_ACC - N_NODES,), NUM_GRAPHS, i32)]
    ).reshape(-1, 128)
    s0 = partials[0].reshape(-1, 128)
    s1 = partials[1].reshape(-1, 128)

    out = pl.pallas_call(
        _tc_finish_body,
        out_shape=jax.ShapeDtypeStruct((NUM_GRAPHS,), jnp.float32),
        out_specs=pl.BlockSpec(memory_space=pltpu.SMEM),
    )(s0, s1, batch_pad)
    return out


# speedup vs baseline: 1.6951x; 1.0010x over previous
import math

import jax
import jax.numpy as jnp
from jax import lax
from jax.experimental import pallas as pl
from jax.experimental.pallas import tpu as pltpu
from jax.experimental.pallas import tpu_sc as plsc

D = 16
R = 1
N_NODES = 100000
N_MOTIFS = 1600000
NUM_TAU = 16
NUM_GRAPHS = 8

NC = 2
NS = 16
NW = NC * NS
MW = N_MOTIFS // NW
B = 1024
NSUB = B // 128
NSLOT = 2
C_RING = 48
TAIL = MW - C_RING * B
TGROUPS = TAIL // 16
GROUPS = B // 16

S_ACC = 100352
SLICE = S_ACC // NS
DUMP = N_NODES

LAMBDA_3 = math.log1p(math.exp(0.5))
BETA_3 = min(math.log1p(math.exp(1.0)), 5.0)
COEF = BETA_3 / math.sqrt(R * D)
OUT_SCALE = LAMBDA_3 / BETA_3


def _sc_body(c_hbm, u_hbm, v_hbm, t_hbm, q_hbm, k_hbm, tt_hbm, z_hbm,
             out_hbm, i_v, q_r, ku_r, kv_r, e_v, csc, tt_v, acc_sp,
             gsem, ssem, isem):
    cid = lax.axis_index("c")
    sid = lax.axis_index("s")
    wid = sid * NC + cid
    mbase = wid * MW

    pltpu.sync_copy(z_hbm, acc_sp.at[pl.ds(sid * SLICE, SLICE)])
    pltpu.sync_copy(tt_hbm, tt_v)
    plsc.subcore_barrier()

    lane = lax.iota(jnp.int32, 16)

    def idx_copies(k, s, n):
        src = pl.ds(mbase + k * B, n)
        dst = pl.ds(0, n)
        return [
            pltpu.make_async_copy(c_hbm.at[src], i_v[s].at[0, dst], isem[s]),
            pltpu.make_async_copy(u_hbm.at[src], i_v[s].at[1, dst], isem[s]),
            pltpu.make_async_copy(v_hbm.at[src], i_v[s].at[2, dst], isem[s]),
            pltpu.make_async_copy(t_hbm.at[src], i_v[s].at[3, dst], isem[s]),
        ]

    def start_idx(k, s, n=B):
        for cp in idx_copies(k, s, n):
            cp.start()

    def wait_idx(k, s, n=B):
        for cp in idx_copies(k, s, n):
            cp.wait()

    def gather_copies(s):
        cps = []
        for j in range(NSUB):
            blk = pl.ds(j * 128, 128)
            cps.append(pltpu.make_async_copy(
                q_hbm.at[i_v[s].at[0, blk]], q_r[s].at[blk], gsem[s]))
            cps.append(pltpu.make_async_copy(
                k_hbm.at[i_v[s].at[1, blk]], ku_r[s].at[blk], gsem[s]))
            cps.append(pltpu.make_async_copy(
                k_hbm.at[i_v[s].at[2, blk]], kv_r[s].at[blk], gsem[s]))
        return cps

    def fire_gathers(s):
        for cp in gather_copies(s):
            cp.start()

    def wait_gathers(s):
        for cp in gather_copies(s):
            cp.wait()

    def scatter_copies(s):
        return [pltpu.make_async_copy(e_v[s].at[j], acc_sp.at[csc[s].at[j]],
                                      ssem[s])
                for j in range(NSUB)]

    def start_scatter(s):
        for cp in scatter_copies(s):
            cp.start(add=True)

    def drain_scatter(s):
        for cp in scatter_copies(s):
            cp.wait()

    def snapshot_c(s, ngroups):
        for p in range(ngroups):
            csc[s][p >> 3, pl.ds((p & 7) * 16, 16)] = (
                i_v[s][0, pl.ds(p * 16, 16)])
        dump = jnp.full((16,), DUMP, jnp.int32)
        for p in range(ngroups, GROUPS):
            csc[s][p >> 3, pl.ds((p & 7) * 16, 16)] = dump

    def compute(s, ngroups):
        qr, kur, kvr, iv, ev = q_r[s], ku_r[s], kv_r[s], i_v[s], e_v[s]

        def group_body(g, carry2):
            m0 = g * 16
            t16 = iv[3, pl.ds(m0, 16)]
            bvec = jnp.zeros((16,), jnp.float32)
            for i in range(16):
                m = m0 + i
                trow = tt_v[t16[i], :]
                w = qr[m, :] * (kur[m, :] * kvr[m, :] + trow)
                bvec = jnp.where(lane == i, jnp.sum(w), bvec)
            e = jnp.exp(bvec * COEF)
            erow = lax.shift_right_logical(g, 3)
            ecol = lax.mul(lax.bitwise_and(g, 7), 16)
            ev[erow, pl.ds(ecol, 16)] = e
            return carry2

        lax.fori_loop(0, ngroups, group_body, 0, unroll=2)

    start_idx(0, 0)
    wait_idx(0, 0)
    fire_gathers(0)
    start_idx(1, 1)

    def outer_body(it, carry):
        for s in range(NSLOT):
            cc = it * NSLOT + s
            wait_gathers(s)

            @pl.when(it > 0)
            def _():
                drain_scatter(s)

            snapshot_c(s, GROUPS)
            compute(s, GROUPS)
            start_scatter(s)

            @pl.when(cc + 2 < C_RING)
            def _():
                start_idx(cc + 2, (s + 2) % NSLOT)

            @pl.when(cc + 1 < C_RING)
            def _():
                wait_idx(cc + 1, (s + 1) % NSLOT)
                fire_gathers((s + 1) % NSLOT)
        return carry

    lax.fori_loop(0, C_RING // NSLOT, outer_body, 0)
    for s in range(NSLOT):
        drain_scatter(s)

    start_idx(C_RING, 0, TAIL)
    wait_idx(C_RING, 0, TAIL)
    fire_gathers(0)
    wait_gathers(0)
    snapshot_c(0, TGROUPS)
    compute(0, TGROUPS)
    start_scatter(0)
    drain_scatter(0)

    plsc.subcore_barrier()
    pltpu.sync_copy(acc_sp.at[pl.ds(sid * SLICE, SLICE)],
                    out_hbm.at[pl.ds(cid * S_ACC + sid * SLICE, SLICE)])


def _tc_finish_body(s0_ref, s1_ref, b_ref, o_ref):
    s = s0_ref[...] + s1_ref[...]
    lse = jnp.where(s > 0.0, jnp.log(s), 0.0)
    for g in range(NUM_GRAPHS):
        eg = jnp.sum(jnp.where(b_ref[...] == g, lse, 0.0))
        o_ref[g] = eg * OUT_SCALE


def kernel(G, c_3, u_3, v_3, t_tau, batch, num_graphs, Q3, K3, T_params,
           num_nodes):
    del G, num_graphs, num_nodes
    i32 = jnp.int32
    c_p = c_3.astype(i32)
    u_p = u_3.astype(i32)
    v_p = v_3.astype(i32)
    t_p = t_tau.astype(i32)
    q2 = Q3.reshape(N_NODES, R * D)
    k2 = K3.reshape(N_NODES, R * D)
    tt = T_params.reshape(NUM_TAU, R * D)
    zeros = jnp.zeros((SLICE,), jnp.float32)

    mesh = plsc.VectorSubcoreMesh(core_axis_name="c", subcore_axis_name="s")
    sc = pl.kernel(
        _sc_body,
        out_type=jax.ShapeDtypeStruct((NC * S_ACC,), jnp.float32),
        mesh=mesh,
        scratch_types=[
            [pltpu.VMEM((4, B), i32) for _ in range(NSLOT)],
            [pltpu.VMEM((B, R * D), jnp.float32) for _ in range(NSLOT)],
            [pltpu.VMEM((B, R * D), jnp.float32) for _ in range(NSLOT)],
            [pltpu.VMEM((B, R * D), jnp.float32) for _ in range(NSLOT)],
            [pltpu.VMEM((NSUB, 128), jnp.float32) for _ in range(NSLOT)],
            [pltpu.VMEM((NSUB, 128), i32) for _ in range(NSLOT)],
            pltpu.VMEM((NUM_TAU, R * D), jnp.float32),
            pltpu.VMEM_SHARED((S_ACC,), jnp.float32),
            [pltpu.SemaphoreType.DMA for _ in range(NSLOT)],
            [pltpu.SemaphoreType.DMA for _ in range(NSLOT)],
            [pltpu.SemaphoreType.DMA for _ in range(NSLOT)],
        ],
        compiler_params=pltpu.CompilerParams(
            needs_layout_passes=False, use_tc_tiling_on_sc=False),
    )
    partials = sc(c_p, u_p, v_p, t_p, q2, k2, tt, zeros).reshape(NC, S_ACC)

    batch_pad = jnp.concatenate(
        [batch.astype(i32), jnp.full((S_ACC - N_NODES,), NUM_GRAPHS, i32)]
    ).reshape(-1, 128)
    s0 = partials[0].reshape(-1, 128)
    s1 = partials[1].reshape(-1, 128)

    out = pl.pallas_call(
        _tc_finish_body,
        out_shape=jax.ShapeDtypeStruct((NUM_GRAPHS,), jnp.float32),
        out_specs=pl.BlockSpec(memory_space=pltpu.SMEM),
    )(s0, s1, batch_pad)
    return out


# speedup vs baseline: 1.6977x; 1.0015x over previous
import math

import jax
import jax.numpy as jnp
from jax import lax
from jax.experimental import pallas as pl
from jax.experimental.pallas import tpu as pltpu
from jax.experimental.pallas import tpu_sc as plsc

D = 16
R = 1
N_NODES = 100000
N_MOTIFS = 1600000
NUM_TAU = 16
NUM_GRAPHS = 8

NC = 2
NS = 16
NW = NC * NS
MW = N_MOTIFS // NW
B = 1024
NSUB = B // 128
NSLOT = 2
C_RING = 48
TAIL = MW - C_RING * B
TGROUPS = TAIL // 16
GROUPS = B // 16

S_ACC = 100352
SLICE = S_ACC // NS
DUMP = N_NODES

LAMBDA_3 = math.log1p(math.exp(0.5))
BETA_3 = min(math.log1p(math.exp(1.0)), 5.0)
COEF = BETA_3 / math.sqrt(R * D)
OUT_SCALE = LAMBDA_3 / BETA_3


def _sc_body(c_hbm, u_hbm, v_hbm, t_hbm, q_hbm, k_hbm, tt_hbm, z_hbm,
             out_hbm, i_v, q_r, ku_r, kv_r, e_v, csc, tt_v, acc_sp,
             gsem, ssem, isem):
    cid = lax.axis_index("c")
    sid = lax.axis_index("s")
    wid = sid * NC + cid
    mbase = wid * MW

    pltpu.sync_copy(z_hbm, acc_sp.at[pl.ds(sid * SLICE, SLICE)])
    pltpu.sync_copy(tt_hbm, tt_v)
    plsc.subcore_barrier()

    lane = lax.iota(jnp.int32, 16)

    def idx_copies(k, s, n):
        src = pl.ds(mbase + k * B, n)
        dst = pl.ds(0, n)
        return [
            pltpu.make_async_copy(c_hbm.at[src], i_v[s].at[0, dst], isem[s]),
            pltpu.make_async_copy(u_hbm.at[src], i_v[s].at[1, dst], isem[s]),
            pltpu.make_async_copy(v_hbm.at[src], i_v[s].at[2, dst], isem[s]),
            pltpu.make_async_copy(t_hbm.at[src], i_v[s].at[3, dst], isem[s]),
        ]

    def start_idx(k, s, n=B):
        for cp in idx_copies(k, s, n):
            cp.start()

    def wait_idx(k, s, n=B):
        for cp in idx_copies(k, s, n):
            cp.wait()

    def gather_copies(s):
        cps = []
        for j in range(NSUB // 2):
            blk = pl.ds(j * 256, 256)
            cps.append(pltpu.make_async_copy(
                q_hbm.at[i_v[s].at[0, blk]], q_r[s].at[blk], gsem[s]))
            cps.append(pltpu.make_async_copy(
                k_hbm.at[i_v[s].at[1, blk]], ku_r[s].at[blk], gsem[s]))
            cps.append(pltpu.make_async_copy(
                k_hbm.at[i_v[s].at[2, blk]], kv_r[s].at[blk], gsem[s]))
        return cps

    def fire_gathers(s):
        for cp in gather_copies(s):
            cp.start()

    def wait_gathers(s):
        for cp in gather_copies(s):
            cp.wait()

    def scatter_copies(s):
        return [pltpu.make_async_copy(e_v[s].at[j], acc_sp.at[csc[s].at[j]],
                                      ssem[s])
                for j in range(NSUB)]

    def start_scatter(s):
        for cp in scatter_copies(s):
            cp.start(add=True)

    def drain_scatter(s):
        for cp in scatter_copies(s):
            cp.wait()

    def snapshot_c(s, ngroups):
        for p in range(ngroups):
            csc[s][p >> 3, pl.ds((p & 7) * 16, 16)] = (
                i_v[s][0, pl.ds(p * 16, 16)])
        dump = jnp.full((16,), DUMP, jnp.int32)
        for p in range(ngroups, GROUPS):
            csc[s][p >> 3, pl.ds((p & 7) * 16, 16)] = dump

    def compute(s, ngroups):
        qr, kur, kvr, iv, ev = q_r[s], ku_r[s], kv_r[s], i_v[s], e_v[s]

        def group_body(g, carry2):
            m0 = g * 16
            t16 = iv[3, pl.ds(m0, 16)]
            bvec = jnp.zeros((16,), jnp.float32)
            for i in range(16):
                m = m0 + i
                trow = tt_v[t16[i], :]
                w = qr[m, :] * (kur[m, :] * kvr[m, :] + trow)
                bvec = jnp.where(lane == i, jnp.sum(w), bvec)
            e = jnp.exp(bvec * COEF)
            erow = lax.shift_right_logical(g, 3)
            ecol = lax.mul(lax.bitwise_and(g, 7), 16)
            ev[erow, pl.ds(ecol, 16)] = e
            return carry2

        lax.fori_loop(0, ngroups, group_body, 0, unroll=2)

    start_idx(0, 0)
    wait_idx(0, 0)
    fire_gathers(0)
    start_idx(1, 1)

    def outer_body(it, carry):
        for s in range(NSLOT):
            cc = it * NSLOT + s
            wait_gathers(s)

            @pl.when(it > 0)
            def _():
                drain_scatter(s)

            snapshot_c(s, GROUPS)
            compute(s, GROUPS)
            start_scatter(s)

            @pl.when(cc + 2 < C_RING)
            def _():
                start_idx(cc + 2, (s + 2) % NSLOT)

            @pl.when(cc + 1 < C_RING)
            def _():
                wait_idx(cc + 1, (s + 1) % NSLOT)
                fire_gathers((s + 1) % NSLOT)
        return carry

    lax.fori_loop(0, C_RING // NSLOT, outer_body, 0)
    for s in range(NSLOT):
        drain_scatter(s)

    start_idx(C_RING, 0, TAIL)
    wait_idx(C_RING, 0, TAIL)
    fire_gathers(0)
    wait_gathers(0)
    snapshot_c(0, TGROUPS)
    compute(0, TGROUPS)
    start_scatter(0)
    drain_scatter(0)

    plsc.subcore_barrier()
    pltpu.sync_copy(acc_sp.at[pl.ds(sid * SLICE, SLICE)],
                    out_hbm.at[pl.ds(cid * S_ACC + sid * SLICE, SLICE)])


def _tc_finish_body(s0_ref, s1_ref, b_ref, o_ref):
    s = s0_ref[...] + s1_ref[...]
    lse = jnp.where(s > 0.0, jnp.log(s), 0.0)
    for g in range(NUM_GRAPHS):
        eg = jnp.sum(jnp.where(b_ref[...] == g, lse, 0.0))
        o_ref[g] = eg * OUT_SCALE


def kernel(G, c_3, u_3, v_3, t_tau, batch, num_graphs, Q3, K3, T_params,
           num_nodes):
    del G, num_graphs, num_nodes
    i32 = jnp.int32
    c_p = c_3.astype(i32)
    u_p = u_3.astype(i32)
    v_p = v_3.astype(i32)
    t_p = t_tau.astype(i32)
    q2 = Q3.reshape(N_NODES, R * D)
    k2 = K3.reshape(N_NODES, R * D)
    tt = T_params.reshape(NUM_TAU, R * D)
    zeros = jnp.zeros((SLICE,), jnp.float32)

    mesh = plsc.VectorSubcoreMesh(core_axis_name="c", subcore_axis_name="s")
    sc = pl.kernel(
        _sc_body,
        out_type=jax.ShapeDtypeStruct((NC * S_ACC,), jnp.float32),
        mesh=mesh,
        scratch_types=[
            [pltpu.VMEM((4, B), i32) for _ in range(NSLOT)],
            [pltpu.VMEM((B, R * D), jnp.float32) for _ in range(NSLOT)],
            [pltpu.VMEM((B, R * D), jnp.float32) for _ in range(NSLOT)],
            [pltpu.VMEM((B, R * D), jnp.float32) for _ in range(NSLOT)],
            [pltpu.VMEM((NSUB, 128), jnp.float32) for _ in range(NSLOT)],
            [pltpu.VMEM((NSUB, 128), i32) for _ in range(NSLOT)],
            pltpu.VMEM((NUM_TAU, R * D), jnp.float32),
            pltpu.VMEM_SHARED((S_ACC,), jnp.float32),
            [pltpu.SemaphoreType.DMA for _ in range(NSLOT)],
            [pltpu.SemaphoreType.DMA for _ in range(NSLOT)],
            [pltpu.SemaphoreType.DMA for _ in range(NSLOT)],
        ],
        compiler_params=pltpu.CompilerParams(
            needs_layout_passes=False, use_tc_tiling_on_sc=False),
    )
    partials = sc(c_p, u_p, v_p, t_p, q2, k2, tt, zeros).reshape(NC, S_ACC)

    batch_pad = jnp.concatenate(
        [batch.astype(i32), jnp.full((S_ACC - N_NODES,), NUM_GRAPHS, i32)]
    ).reshape(-1, 128)
    s0 = partials[0].reshape(-1, 128)
    s1 = partials[1].reshape(-1, 128)

    out = pl.pallas_call(
        _tc_finish_body,
        out_shape=jax.ShapeDtypeStruct((NUM_GRAPHS,), jnp.float32),
        out_specs=pl.BlockSpec(memory_space=pltpu.SMEM),
    )(s0, s1, batch_pad)
    return out
